# initial kernel scaffold (unmeasured)
import jax
import jax.numpy as jnp
from jax import lax
from jax.experimental import pallas as pl
from jax.experimental.pallas import tpu as pltpu

N_GLOBAL = 4096
EPS = 1e-5
TILE = 512


def kernel(x, gamma, beta):
    m, n = x.shape
    n_tiles = m // TILE

    def body(x_hbm, gamma_ref, beta_ref, out_hbm,
             in_tile, out_tile, my_stats, peer_stats,
             in_sem, out_sem, send_sem, recv_sem):
        my_x = lax.axis_index("x")
        my_y = lax.axis_index("y")

        for t in range(n_tiles):
            cp = pltpu.make_async_copy(
                x_hbm.at[pl.ds(t * TILE, TILE), :], in_tile, in_sem)
            cp.start()
            cp.wait()
            tile = in_tile[:, :]
            s = jnp.sum(tile, axis=1, keepdims=True)
            sq = jnp.sum(tile * tile, axis=1, keepdims=True)
            my_stats[pl.ds(t * TILE, TILE), :] = jnp.concatenate(
                [s, sq], axis=1)

        rdma = pltpu.make_async_remote_copy(
            src_ref=my_stats,
            dst_ref=peer_stats,
            send_sem=send_sem,
            recv_sem=recv_sem,
            device_id=(my_x, 1 - my_y),
            device_id_type=pl.DeviceIdType.MESH,
        )
        rdma.start()
        rdma.wait()

        total = my_stats[:, :] + peer_stats[:, :]
        mean = total[:, 0:1] * (1.0 / N_GLOBAL)
        ex2 = total[:, 1:2] * (1.0 / N_GLOBAL)
        rstd = lax.rsqrt(ex2 - mean * mean + EPS)
        g = gamma_ref[:, :]
        b = beta_ref[:, :]

        for t in range(n_tiles):
            cp = pltpu.make_async_copy(
                x_hbm.at[pl.ds(t * TILE, TILE), :], in_tile, in_sem)
            cp.start()
            cp.wait()
            m_t = mean[t * TILE:(t + 1) * TILE]
            r_t = rstd[t * TILE:(t + 1) * TILE]
            out_tile[:, :] = (in_tile[:, :] - m_t) * r_t * g + b
            cpo = pltpu.make_async_copy(
                out_tile, out_hbm.at[pl.ds(t * TILE, TILE), :], out_sem)
            cpo.start()
            cpo.wait()

    return pl.pallas_call(
        body,
        out_shape=jax.ShapeDtypeStruct((m, n), jnp.float32),
        in_specs=[
            pl.BlockSpec(memory_space=pltpu.ANY),
            pl.BlockSpec(memory_space=pltpu.VMEM),
            pl.BlockSpec(memory_space=pltpu.VMEM),
        ],
        out_specs=pl.BlockSpec(memory_space=pltpu.ANY),
        scratch_shapes=[
            pltpu.VMEM((TILE, n), jnp.float32),
            pltpu.VMEM((TILE, n), jnp.float32),
            pltpu.VMEM((m, 2), jnp.float32),
            pltpu.VMEM((m, 2), jnp.float32),
            pltpu.SemaphoreType.DMA,
            pltpu.SemaphoreType.DMA,
            pltpu.SemaphoreType.DMA,
            pltpu.SemaphoreType.DMA,
        ],
        compiler_params=pltpu.CompilerParams(collective_id=0),
    )(x, gamma.reshape(1, n), beta.reshape(1, n))


# baseline (device time: 216767 ns/iter reference)
import jax
import jax.numpy as jnp
from jax import lax
from jax.experimental import pallas as pl
from jax.experimental.pallas import tpu as pltpu

N_GLOBAL = 4096
EPS = 1e-5
TILE = 512


def kernel(x, gamma, beta):
    m, n = x.shape
    n_tiles = m // TILE

    def body(x_hbm, gamma_ref, beta_ref, out_hbm,
             in_tile, out_tile, my_stats, peer_stats,
             in_sem, out_sem, send_sem, recv_sem):
        my_x = lax.axis_index("x")
        my_y = lax.axis_index("y")

        barrier_sem = pltpu.get_barrier_semaphore()
        pl.semaphore_signal(
            barrier_sem, inc=1,
            device_id=(my_x, 1 - my_y),
            device_id_type=pl.DeviceIdType.MESH,
        )
        pl.semaphore_wait(barrier_sem, 1)

        for t in range(n_tiles):
            cp = pltpu.make_async_copy(
                x_hbm.at[pl.ds(t * TILE, TILE), :], in_tile, in_sem)
            cp.start()
            cp.wait()
            tile = in_tile[:, :]
            s = jnp.sum(tile, axis=1, keepdims=True)
            sq = jnp.sum(tile * tile, axis=1, keepdims=True)
            my_stats[pl.ds(t * TILE, TILE), :] = jnp.concatenate(
                [s, sq], axis=1)

        rdma = pltpu.make_async_remote_copy(
            src_ref=my_stats,
            dst_ref=peer_stats,
            send_sem=send_sem,
            recv_sem=recv_sem,
            device_id=(my_x, 1 - my_y),
            device_id_type=pl.DeviceIdType.MESH,
        )
        rdma.start()
        rdma.wait()

        total = my_stats[:, :] + peer_stats[:, :]
        mean = total[:, 0:1] * (1.0 / N_GLOBAL)
        ex2 = total[:, 1:2] * (1.0 / N_GLOBAL)
        rstd = lax.rsqrt(ex2 - mean * mean + EPS)
        g = gamma_ref[:, :]
        b = beta_ref[:, :]

        for t in range(n_tiles):
            cp = pltpu.make_async_copy(
                x_hbm.at[pl.ds(t * TILE, TILE), :], in_tile, in_sem)
            cp.start()
            cp.wait()
            m_t = mean[t * TILE:(t + 1) * TILE]
            r_t = rstd[t * TILE:(t + 1) * TILE]
            out_tile[:, :] = (in_tile[:, :] - m_t) * r_t * g + b
            cpo = pltpu.make_async_copy(
                out_tile, out_hbm.at[pl.ds(t * TILE, TILE), :], out_sem)
            cpo.start()
            cpo.wait()

    return pl.pallas_call(
        body,
        out_shape=jax.ShapeDtypeStruct((m, n), jnp.float32),
        in_specs=[
            pl.BlockSpec(memory_space=pl.ANY),
            pl.BlockSpec(memory_space=pltpu.VMEM),
            pl.BlockSpec(memory_space=pltpu.VMEM),
        ],
        out_specs=pl.BlockSpec(memory_space=pl.ANY),
        scratch_shapes=[
            pltpu.VMEM((TILE, n), jnp.float32),
            pltpu.VMEM((TILE, n), jnp.float32),
            pltpu.VMEM((m, 2), jnp.float32),
            pltpu.VMEM((m, 2), jnp.float32),
            pltpu.SemaphoreType.DMA,
            pltpu.SemaphoreType.DMA,
            pltpu.SemaphoreType.DMA,
            pltpu.SemaphoreType.DMA,
        ],
        compiler_params=pltpu.CompilerParams(collective_id=0),
    )(x, gamma.reshape(1, n), beta.reshape(1, n))


# device time: 107232 ns/iter; 2.0215x vs baseline; 2.0215x over previous
import jax
import jax.numpy as jnp
from jax import lax
from jax.experimental import pallas as pl
from jax.experimental.pallas import tpu as pltpu

N_GLOBAL = 4096
EPS = 1e-5
TILE = 512


def kernel(x, gamma, beta):
    m, n = x.shape
    n_tiles = m // TILE

    def body(x_hbm, gamma_ref, beta_ref, out_hbm,
             in_tiles, out_tiles, my_stats, peer_stats,
             in_sems, out_sems, send_sems, recv_sems):
        my_x = lax.axis_index("x")
        my_y = lax.axis_index("y")
        peer = (my_x, 1 - my_y)

        barrier_sem = pltpu.get_barrier_semaphore()
        pl.semaphore_signal(
            barrier_sem, inc=1,
            device_id=peer, device_id_type=pl.DeviceIdType.MESH,
        )
        pl.semaphore_wait(barrier_sem, 1)

        g = gamma_ref[:, :]
        b = beta_ref[:, :]

        def in_copy(t):
            return pltpu.make_async_copy(
                x_hbm.at[pl.ds(t * TILE, TILE), :],
                in_tiles.at[t % 3], in_sems.at[t % 3])

        def rdma_for(t):
            return pltpu.make_async_remote_copy(
                src_ref=my_stats.at[pl.ds(t * TILE, TILE), :],
                dst_ref=peer_stats.at[pl.ds(t * TILE, TILE), :],
                send_sem=send_sems.at[t],
                recv_sem=recv_sems.at[t],
                device_id=peer, device_id_type=pl.DeviceIdType.MESH)

        out_copies = [None] * n_tiles

        def finish(u):
            if u >= 2:
                out_copies[u - 2].wait()
            r = rdma_for(u)
            r.wait_send()
            r.wait_recv()
            tot = (my_stats[u * TILE:(u + 1) * TILE, :]
                   + peer_stats[u * TILE:(u + 1) * TILE, :])
            mean = tot[:, 0:1] * (1.0 / N_GLOBAL)
            rstd = lax.rsqrt(tot[:, 1:2] * (1.0 / N_GLOBAL)
                             - mean * mean + EPS)
            out_tiles[u % 2] = (in_tiles[u % 3] - mean) * rstd * g + b
            cpo = pltpu.make_async_copy(
                out_tiles.at[u % 2],
                out_hbm.at[pl.ds(u * TILE, TILE), :], out_sems.at[u % 2])
            cpo.start()
            out_copies[u] = cpo

        in_copies = {0: in_copy(0)}
        in_copies[0].start()
        for t in range(n_tiles):
            in_copies[t].wait()
            if t + 1 < n_tiles:
                in_copies[t + 1] = in_copy(t + 1)
                in_copies[t + 1].start()
            tile = in_tiles[t % 3]
            s = jnp.sum(tile, axis=1, keepdims=True)
            sq = jnp.sum(tile * tile, axis=1, keepdims=True)
            my_stats[pl.ds(t * TILE, TILE), :] = jnp.concatenate(
                [s, sq], axis=1)
            rdma_for(t).start()
            if t >= 1:
                finish(t - 1)
        finish(n_tiles - 1)
        out_copies[n_tiles - 2].wait()
        out_copies[n_tiles - 1].wait()

    return pl.pallas_call(
        body,
        out_shape=jax.ShapeDtypeStruct((m, n), jnp.float32),
        in_specs=[
            pl.BlockSpec(memory_space=pl.ANY),
            pl.BlockSpec(memory_space=pltpu.VMEM),
            pl.BlockSpec(memory_space=pltpu.VMEM),
        ],
        out_specs=pl.BlockSpec(memory_space=pl.ANY),
        scratch_shapes=[
            pltpu.VMEM((3, TILE, n), jnp.float32),
            pltpu.VMEM((2, TILE, n), jnp.float32),
            pltpu.VMEM((m, 2), jnp.float32),
            pltpu.VMEM((m, 2), jnp.float32),
            pltpu.SemaphoreType.DMA((3,)),
            pltpu.SemaphoreType.DMA((2,)),
            pltpu.SemaphoreType.DMA((m // TILE,)),
            pltpu.SemaphoreType.DMA((m // TILE,)),
        ],
        compiler_params=pltpu.CompilerParams(collective_id=0),
    )(x, gamma.reshape(1, n), beta.reshape(1, n))


# device time: 97387 ns/iter; 2.2258x vs baseline; 1.1011x over previous
import jax
import jax.numpy as jnp
from jax import lax
from jax.experimental import pallas as pl
from jax.experimental.pallas import tpu as pltpu

N_GLOBAL = 4096
EPS = 1e-5
TILE = 1024


def kernel(x, gamma, beta):
    m, n = x.shape
    n_tiles = m // TILE

    def body(x_hbm, gamma_ref, beta_ref, out_hbm,
             in_tiles, out_tiles, my_stats, peer_stats,
             in_sems, out_sems, send_sems, recv_sems):
        my_x = lax.axis_index("x")
        my_y = lax.axis_index("y")
        peer = (my_x, 1 - my_y)


        g = gamma_ref[:, :]
        b = beta_ref[:, :]

        def in_copy(t):
            return pltpu.make_async_copy(
                x_hbm.at[pl.ds(t * TILE, TILE), :],
                in_tiles.at[t % 3], in_sems.at[t % 3])

        def rdma_for(t):
            return pltpu.make_async_remote_copy(
                src_ref=my_stats.at[t],
                dst_ref=peer_stats.at[t],
                send_sem=send_sems.at[t],
                recv_sem=recv_sems.at[t],
                device_id=peer, device_id_type=pl.DeviceIdType.MESH)

        out_copies = [None] * n_tiles

        def finish(u):
            if u >= 2:
                out_copies[u - 2].wait()
            r = rdma_for(u)
            r.wait_send()
            r.wait_recv()
            tot = my_stats[u] + peer_stats[u]
            mean_p = tot[0] * (1.0 / N_GLOBAL)
            ex2_p = tot[1] * (1.0 / N_GLOBAL)
            rstd_p = lax.rsqrt(ex2_p - mean_p * mean_p + EPS)
            mr_p = mean_p * rstd_p
            rstd_3 = rstd_p.reshape(8, TILE // 8, 1)
            mr_3 = mr_p.reshape(8, TILE // 8, 1)
            tile3 = in_tiles[u % 3].reshape(8, TILE // 8, n)
            out3 = (tile3 * rstd_3 - mr_3) * g.reshape(1, 1, n) + \
                b.reshape(1, 1, n)
            out_tiles[u % 2] = out3.reshape(TILE, n)
            cpo = pltpu.make_async_copy(
                out_tiles.at[u % 2],
                out_hbm.at[pl.ds(u * TILE, TILE), :], out_sems.at[u % 2])
            cpo.start()
            out_copies[u] = cpo

        in_copies = {0: in_copy(0)}
        in_copies[0].start()
        for t in range(n_tiles):
            in_copies[t].wait()
            if t + 1 < n_tiles:
                in_copies[t + 1] = in_copy(t + 1)
                in_copies[t + 1].start()
            tile = in_tiles[t % 3]
            t3 = tile.reshape(8, TILE // 8, n)
            s = jnp.sum(t3, axis=2)
            sq = jnp.sum(t3 * t3, axis=2)
            my_stats[t, 0] = s
            my_stats[t, 1] = sq
            rdma_for(t).start()
            if t >= 1:
                finish(t - 1)
        finish(n_tiles - 1)
        out_copies[n_tiles - 2].wait()
        out_copies[n_tiles - 1].wait()

    return pl.pallas_call(
        body,
        out_shape=jax.ShapeDtypeStruct((m, n), jnp.float32),
        in_specs=[
            pl.BlockSpec(memory_space=pl.ANY),
            pl.BlockSpec(memory_space=pltpu.VMEM),
            pl.BlockSpec(memory_space=pltpu.VMEM),
        ],
        out_specs=pl.BlockSpec(memory_space=pl.ANY),
        scratch_shapes=[
            pltpu.VMEM((3, TILE, n), jnp.float32),
            pltpu.VMEM((2, TILE, n), jnp.float32),
            pltpu.VMEM((m // TILE, 2, 8, TILE // 8), jnp.float32),
            pltpu.VMEM((m // TILE, 2, 8, TILE // 8), jnp.float32),
            pltpu.SemaphoreType.DMA((3,)),
            pltpu.SemaphoreType.DMA((2,)),
            pltpu.SemaphoreType.DMA((m // TILE,)),
            pltpu.SemaphoreType.DMA((m // TILE,)),
        ],
        compiler_params=pltpu.CompilerParams(
            vmem_limit_bytes=100 * 1024 * 1024),
    )(x, gamma.reshape(1, n), beta.reshape(1, n))
